# baseline (speedup 1.0000x reference)
import functools
import math

import jax
import jax.numpy as jnp
from jax import lax
from jax.experimental import pallas as pl
from jax.experimental.pallas import tpu as pltpu
from jax.experimental.pallas import tpu_sc as plsc

D_MODEL = 64
SCALE = math.sqrt(D_MODEL)

_info = plsc.get_sparse_core_info()
_NC, _NS, _L = _info.num_cores, _info.num_subcores, _info.num_lanes
_NW = _NC * _NS

BLK = 128
NBUF = 4
LOOKAHEAD = 3


def _sc_embed(table_pairs, xt):
  n_seq, n_batch = xt.shape
  assert n_batch == _NW * BLK
  n_blocks = n_seq
  assert n_blocks % NBUF == 0

  mesh = plsc.VectorSubcoreMesh(core_axis_name="c", subcore_axis_name="s")

  @functools.partial(
      pl.kernel,
      mesh=mesh,
      out_type=jax.ShapeDtypeStruct((n_seq, D_MODEL, n_batch), jnp.float32),
      scratch_types=[
          pltpu.VMEM((n_seq + 1, BLK), jnp.int32),
      ] + [pltpu.VMEM((BLK,), jnp.int32)] * NBUF
        + [pltpu.VMEM((BLK, BLK), jnp.float32)] * NBUF
        + [pltpu.VMEM((D_MODEL, BLK + 1), jnp.float32)] * 2
        + [pltpu.SemaphoreType.DMA] * (1 + NBUF + 2),
      compiler_params=pltpu.CompilerParams(
          use_tc_tiling_on_sc=True, needs_layout_passes=False),
  )
  def k(tp_hbm, xt_hbm, out_hbm, idx_all, *rest):
    pairb = rest[:NBUF]
    rows = rest[NBUF:2 * NBUF]
    blk = rest[2 * NBUF:2 * NBUF + 2]
    isem = rest[2 * NBUF + 2]
    gsem = rest[2 * NBUF + 3:2 * NBUF + 3 + NBUF]
    ssem = rest[2 * NBUF + 3 + NBUF:]

    w = lax.axis_index("s") * _NC + lax.axis_index("c")
    col0 = w * BLK

    idx_copies = []
    for sr in range(n_seq // 8):
      idx_copies.append(
          pltpu.async_copy(
              xt_hbm.at[pl.ds(sr * 8, 8), pl.ds(col0, BLK)],
              idx_all.at[pl.ds(sr * 8, 8)],
              isem,
          ))
    for c in idx_copies:
      c.wait()

    def fire(m, b):
      for g in range(BLK // _L):
        v = idx_all[m, pl.ds(g * _L, _L)]
        pairb[b][pl.ds(g * _L, _L)] = v >> 1
      pltpu.async_copy(tp_hbm.at[pairb[b]], rows[b], gsem[b])

    def drain_store(b, s):
      pltpu.make_async_copy(
          blk[b].at[:, pl.ds(0, BLK)],
          out_hbm.at[s, :, pl.ds(col0, BLK)], ssem[b]).wait()

    dvecs = [lax.iota(jnp.int32, _L) + j * _L for j in range(D_MODEL // _L)]

    def process(ci, b):
      b2 = b % 2
      pltpu.make_async_copy(tp_hbm.at[pairb[b]], rows[b], gsem[b]).wait()

      @pl.when(ci >= 2)
      def _():
        drain_store(b2, 0)

      def tbody(t, carry):
        h64 = ((idx_all[ci, pl.ds(t, _L)])[0] & 1) << 6
        vals = [
            rows[b][t, pl.ds(h64 + j * _L, _L)]
            for j in range(D_MODEL // _L)
        ]
        tvec = jnp.full((_L,), t, dtype=jnp.int32)
        for j in range(D_MODEL // _L):
          plsc.store_scatter(blk[b2], [dvecs[j], tvec], vals[j] * SCALE)
        return carry

      lax.fori_loop(0, BLK, tbody, 0, unroll=4)
      pltpu.async_copy(
          blk[b2].at[:, pl.ds(0, BLK)],
          out_hbm.at[ci, :, pl.ds(col0, BLK)], ssem[b2])

    for m in range(LOOKAHEAD):
      fire(m, m % NBUF)

    def group_body(g, carry):
      for b in range(NBUF):
        ci = g * NBUF + b
        m = ci + LOOKAHEAD
        bm = (b + LOOKAHEAD) % NBUF

        @pl.when(m < n_blocks)
        def _():
          fire(m, bm)

        process(ci, b)
      return carry

    lax.fori_loop(0, n_blocks // NBUF, group_body, 0)

    for b in range(2):
      drain_store(b, 0)

  return k(table_pairs, xt)


def kernel(x, table):
  b, s = x.shape
  table_pairs = table.reshape(table.shape[0] // 2, 2 * D_MODEL)
  xt = jnp.transpose(x).astype(jnp.int32)
  outT = _sc_embed(table_pairs, xt)
  return jnp.transpose(outT, (2, 0, 1))


# speedup vs baseline: 1.3562x; 1.3562x over previous
import functools
import math

import jax
import jax.numpy as jnp
from jax import lax
from jax.experimental import pallas as pl
from jax.experimental.pallas import tpu as pltpu
from jax.experimental.pallas import tpu_sc as plsc

D_MODEL = 64
SCALE = math.sqrt(D_MODEL)

_info = plsc.get_sparse_core_info()
_NC, _NS, _L = _info.num_cores, _info.num_subcores, _info.num_lanes
_NW = _NC * _NS

BLK = 128
NBUF = 4
LOOKAHEAD = 2

PAIR_W = 2 * D_MODEL


def _sc_gather_pairs(table_pairs, xt):
  n_seq, n_batch = xt.shape
  assert n_batch == _NW * BLK
  n_blocks = n_seq
  assert n_blocks % NBUF == 0

  mesh = plsc.VectorSubcoreMesh(core_axis_name="c", subcore_axis_name="s")

  @functools.partial(
      pl.kernel,
      mesh=mesh,
      out_type=jax.ShapeDtypeStruct((n_seq, n_batch, PAIR_W), jnp.float32),
      scratch_types=[
          pltpu.VMEM((n_seq, BLK), jnp.int32),
      ] + [pltpu.VMEM((BLK,), jnp.int32)] * NBUF
        + [pltpu.VMEM((BLK, PAIR_W), jnp.float32)] * NBUF
        + [pltpu.SemaphoreType.DMA] * (1 + 2 * NBUF),
      compiler_params=pltpu.CompilerParams(
          use_tc_tiling_on_sc=True, needs_layout_passes=False),
  )
  def k(tp_hbm, xt_hbm, out_hbm, idx_all, *rest):
    pairb = rest[:NBUF]
    rows = rest[NBUF:2 * NBUF]
    isem = rest[2 * NBUF]
    gsem = rest[2 * NBUF + 1:2 * NBUF + 1 + NBUF]
    ssem = rest[2 * NBUF + 1 + NBUF:]

    w = lax.axis_index("s") * _NC + lax.axis_index("c")
    col0 = w * BLK

    idx_copies = []
    for sr in range(n_seq // 8):
      idx_copies.append(
          pltpu.async_copy(
              xt_hbm.at[pl.ds(sr * 8, 8), pl.ds(col0, BLK)],
              idx_all.at[pl.ds(sr * 8, 8)],
              isem,
          ))
    for c in idx_copies:
      c.wait()

    def drain_store(b):
      pltpu.make_async_copy(
          rows[b], out_hbm.at[0, pl.ds(col0, BLK), :], ssem[b]).wait()

    def fire(m, b):
      @pl.when(m >= NBUF)
      def _():
        drain_store(b)

      for g in range(BLK // _L):
        v = idx_all[m, pl.ds(g * _L, _L)]
        pairb[b][pl.ds(g * _L, _L)] = v >> 1
      pltpu.async_copy(tp_hbm.at[pairb[b]], rows[b], gsem[b])

    def process(ci, b):
      pltpu.make_async_copy(tp_hbm.at[pairb[b]], rows[b], gsem[b]).wait()
      pltpu.async_copy(
          rows[b], out_hbm.at[ci, pl.ds(col0, BLK), :], ssem[b])

    for m in range(LOOKAHEAD):
      fire(m, m % NBUF)

    def group_body(g, carry):
      for b in range(NBUF):
        ci = g * NBUF + b
        m = ci + LOOKAHEAD
        bm = (b + LOOKAHEAD) % NBUF

        @pl.when(m < n_blocks)
        def _():
          fire(m, bm)

        process(ci, b)
      return carry

    lax.fori_loop(0, n_blocks // NBUF, group_body, 0)

    for b in range(NBUF):
      drain_store(b)

  return k(table_pairs, xt)


def kernel(x, table):
  table_pairs = table.reshape(table.shape[0] // 2, PAIR_W)
  xt = jnp.transpose(x).astype(jnp.int32)
  raw = _sc_gather_pairs(table_pairs, xt)
  odd = (xt & 1).astype(bool)
  sel = jnp.where(odd[:, :, None], raw[:, :, D_MODEL:], raw[:, :, :D_MODEL])
  out = sel * SCALE
  return jnp.transpose(out, (1, 0, 2))
